# baseline (speedup 1.0000x reference)
import jax
import jax.numpy as jnp
from jax import lax
from jax.experimental import pallas as pl
from jax.experimental.pallas import tpu as pltpu
from jax.experimental.pallas import tpu_sc as plsc


def kernel(indices, E):
    B, H = indices.shape
    V, D = E.shape
    E_T = E.T
    idx_T = indices.T

    mesh = plsc.VectorSubcoreMesh(core_axis_name="core",
                                  subcore_axis_name="subcore")
    n_sub = 32
    d_per = D // n_sub

    @pl.kernel(
        out_type=jax.ShapeDtypeStruct((H, D, B), E.dtype),
        mesh=mesh,
        scratch_types=[
            pltpu.VMEM((V,), E.dtype),
            pltpu.VMEM((B,), indices.dtype),
            pltpu.VMEM((B,), E.dtype),
            pltpu.SemaphoreType.DMA,
        ],
        compiler_params=pltpu.CompilerParams(use_tc_tiling_on_sc=False,
                                             needs_layout_passes=False),
    )
    def gather_kernel(et_hbm, it_hbm, o_hbm, row, idxb, outb, sem):
        c = lax.axis_index("core")
        s = lax.axis_index("subcore")
        t = c * 16 + s

        @pl.loop(0, d_per)
        def _(j):
            d = t * d_per + j
            pltpu.async_copy(et_hbm.at[d], row, sem).wait()

            @pl.loop(0, H)
            def _(h):
                pltpu.async_copy(it_hbm.at[h], idxb, sem).wait()

                @pl.loop(0, B // 16)
                def _(k):
                    iv = idxb[pl.ds(k * 16, 16)]
                    outb[pl.ds(k * 16, 16)] = plsc.load_gather(row, [iv])

                pltpu.async_copy(outb, o_hbm.at[h, d], sem).wait()

    out = gather_kernel(E_T, idx_T)
    return jnp.transpose(out, (2, 0, 1))


# speedup vs baseline: 1.5986x; 1.5986x over previous
import jax
import jax.numpy as jnp
from jax import lax
from jax.experimental import pallas as pl
from jax.experimental.pallas import tpu as pltpu
from jax.experimental.pallas import tpu_sc as plsc

_LANES = 16
_UNROLL = 8


def kernel(indices, E):
    B, H = indices.shape
    V, D = E.shape
    E_T = E.T
    idx_T = indices.T

    mesh = plsc.VectorSubcoreMesh(core_axis_name="core",
                                  subcore_axis_name="subcore")
    n_sub = 32
    d_per = D // n_sub

    @pl.kernel(
        out_type=jax.ShapeDtypeStruct((H, D, B), E.dtype),
        mesh=mesh,
        scratch_types=[
            pltpu.VMEM((V,), E.dtype),
            pltpu.VMEM((B,), indices.dtype),
            pltpu.VMEM((B,), indices.dtype),
            pltpu.VMEM((B,), E.dtype),
            pltpu.VMEM((B,), E.dtype),
            pltpu.SemaphoreType.DMA,
            pltpu.SemaphoreType.DMA,
            pltpu.SemaphoreType.DMA,
            pltpu.SemaphoreType.DMA,
            pltpu.SemaphoreType.DMA,
        ],
        compiler_params=pltpu.CompilerParams(use_tc_tiling_on_sc=False,
                                             needs_layout_passes=False),
    )
    def gather_kernel(et_hbm, it_hbm, o_hbm, row, ib0, ib1, ob0, ob1,
                      sem_row, sem_i0, sem_i1, sem_o0, sem_o1):
        c = lax.axis_index("core")
        s = lax.axis_index("subcore")
        t = c * 16 + s

        def gather_into(ob, ib):
            @pl.loop(0, B // (_LANES * _UNROLL))
            def _(k):
                base = k * (_LANES * _UNROLL)
                for u in range(_UNROLL):
                    sl = pl.ds(base + u * _LANES, _LANES)
                    ob[sl] = plsc.load_gather(row, [ib[sl]])

        @pl.loop(0, d_per)
        def _(j):
            d = t * d_per + j
            pltpu.make_async_copy(et_hbm.at[d], row, sem_row).start()
            pltpu.make_async_copy(it_hbm.at[0], ib0, sem_i0).start()
            pltpu.make_async_copy(it_hbm.at[1], ib1, sem_i1).start()
            pltpu.make_async_copy(et_hbm.at[d], row, sem_row).wait()

            @pl.loop(0, H // 2)
            def _(hh):
                h0 = 2 * hh
                h1 = h0 + 1

                pltpu.make_async_copy(it_hbm.at[h0], ib0, sem_i0).wait()

                @pl.when(hh > 0)
                def _():
                    pltpu.make_async_copy(ob0, o_hbm.at[h0 - 2, d],
                                          sem_o0).wait()

                gather_into(ob0, ib0)
                pltpu.make_async_copy(ob0, o_hbm.at[h0, d], sem_o0).start()

                @pl.when(h0 + 2 < H)
                def _():
                    pltpu.make_async_copy(it_hbm.at[h0 + 2], ib0,
                                          sem_i0).start()

                pltpu.make_async_copy(it_hbm.at[h1], ib1, sem_i1).wait()

                @pl.when(hh > 0)
                def _():
                    pltpu.make_async_copy(ob1, o_hbm.at[h1 - 2, d],
                                          sem_o1).wait()

                gather_into(ob1, ib1)
                pltpu.make_async_copy(ob1, o_hbm.at[h1, d], sem_o1).start()

                @pl.when(h1 + 2 < H)
                def _():
                    pltpu.make_async_copy(it_hbm.at[h1 + 2], ib1,
                                          sem_i1).start()

            pltpu.make_async_copy(ob0, o_hbm.at[H - 2, d], sem_o0).wait()
            pltpu.make_async_copy(ob1, o_hbm.at[H - 1, d], sem_o1).wait()

    out = gather_kernel(E_T, idx_T)
    return jnp.transpose(out, (2, 0, 1))


# speedup vs baseline: 1.8786x; 1.1752x over previous
import jax
import jax.numpy as jnp
from jax import lax
from jax.experimental import pallas as pl
from jax.experimental.pallas import tpu as pltpu
from jax.experimental.pallas import tpu_sc as plsc

_LANES = 16
_UNROLL = 8


def kernel(indices, E):
    B, H = indices.shape
    V, D = E.shape
    E_T = E.T
    idx_T = indices.T

    mesh = plsc.VectorSubcoreMesh(core_axis_name="core",
                                  subcore_axis_name="subcore")
    n_sub = 32
    d_per = D // n_sub

    @pl.kernel(
        out_type=jax.ShapeDtypeStruct((H, D, B), E.dtype),
        mesh=mesh,
        scratch_types=[
            pltpu.VMEM((V,), E.dtype),
            pltpu.VMEM((B,), indices.dtype),
            pltpu.VMEM((B,), indices.dtype),
            pltpu.VMEM((B,), E.dtype),
            pltpu.VMEM((B,), E.dtype),
            pltpu.SemaphoreType.DMA,
            pltpu.SemaphoreType.DMA,
            pltpu.SemaphoreType.DMA,
            pltpu.SemaphoreType.DMA,
            pltpu.SemaphoreType.DMA,
        ],
        compiler_params=pltpu.CompilerParams(use_tc_tiling_on_sc=False,
                                             needs_layout_passes=False),
    )
    def gather_kernel(et_hbm, it_hbm, o_hbm, row, ib0, ib1, ob0, ob1,
                      sem_row, sem_i0, sem_i1, sem_o0, sem_o1):
        c = lax.axis_index("core")
        s = lax.axis_index("subcore")
        t = c * 16 + s

        def gather_into(ob, ib):
            @plsc.parallel_loop(0, B, step=_LANES, unroll=_UNROLL)
            def _(i):
                sl = pl.ds(i, _LANES)
                ob[sl] = plsc.load_gather(row, [ib[sl]])

        @pl.loop(0, d_per)
        def _(j):
            d = t * d_per + j
            pltpu.make_async_copy(et_hbm.at[d], row, sem_row).start()
            pltpu.make_async_copy(it_hbm.at[0], ib0, sem_i0).start()
            pltpu.make_async_copy(it_hbm.at[1], ib1, sem_i1).start()
            pltpu.make_async_copy(et_hbm.at[d], row, sem_row).wait()

            @pl.loop(0, H // 2)
            def _(hh):
                h0 = 2 * hh
                h1 = h0 + 1

                pltpu.make_async_copy(it_hbm.at[h0], ib0, sem_i0).wait()

                @pl.when(hh > 0)
                def _():
                    pltpu.make_async_copy(ob0, o_hbm.at[h0 - 2, d],
                                          sem_o0).wait()

                gather_into(ob0, ib0)
                pltpu.make_async_copy(ob0, o_hbm.at[h0, d], sem_o0).start()

                @pl.when(h0 + 2 < H)
                def _():
                    pltpu.make_async_copy(it_hbm.at[h0 + 2], ib0,
                                          sem_i0).start()

                pltpu.make_async_copy(it_hbm.at[h1], ib1, sem_i1).wait()

                @pl.when(hh > 0)
                def _():
                    pltpu.make_async_copy(ob1, o_hbm.at[h1 - 2, d],
                                          sem_o1).wait()

                gather_into(ob1, ib1)
                pltpu.make_async_copy(ob1, o_hbm.at[h1, d], sem_o1).start()

                @pl.when(h1 + 2 < H)
                def _():
                    pltpu.make_async_copy(it_hbm.at[h1 + 2], ib1,
                                          sem_i1).start()

            pltpu.make_async_copy(ob0, o_hbm.at[H - 2, d], sem_o0).wait()
            pltpu.make_async_copy(ob1, o_hbm.at[H - 1, d], sem_o1).wait()

    out = gather_kernel(E_T, idx_T)
    return jnp.transpose(out, (2, 0, 1))


# speedup vs baseline: 1.8793x; 1.0004x over previous
import jax
import jax.numpy as jnp
from jax import lax
from jax.experimental import pallas as pl
from jax.experimental.pallas import tpu as pltpu
from jax.experimental.pallas import tpu_sc as plsc

_LANES = 16
_UNROLL = 16


def kernel(indices, E):
    B, H = indices.shape
    V, D = E.shape
    E_T = E.T
    idx_T = indices.T

    mesh = plsc.VectorSubcoreMesh(core_axis_name="core",
                                  subcore_axis_name="subcore")
    n_sub = 32
    d_per = D // n_sub

    @pl.kernel(
        out_type=jax.ShapeDtypeStruct((H, D, B), E.dtype),
        mesh=mesh,
        scratch_types=[
            pltpu.VMEM((V,), E.dtype),
            pltpu.VMEM((B,), indices.dtype),
            pltpu.VMEM((B,), indices.dtype),
            pltpu.VMEM((B,), E.dtype),
            pltpu.VMEM((B,), E.dtype),
            pltpu.SemaphoreType.DMA,
            pltpu.SemaphoreType.DMA,
            pltpu.SemaphoreType.DMA,
            pltpu.SemaphoreType.DMA,
            pltpu.SemaphoreType.DMA,
        ],
        compiler_params=pltpu.CompilerParams(use_tc_tiling_on_sc=False,
                                             needs_layout_passes=False),
    )
    def gather_kernel(et_hbm, it_hbm, o_hbm, row, ib0, ib1, ob0, ob1,
                      sem_row, sem_i0, sem_i1, sem_o0, sem_o1):
        c = lax.axis_index("core")
        s = lax.axis_index("subcore")
        t = c * 16 + s

        def gather_into(ob, ib):
            @plsc.parallel_loop(0, B, step=_LANES, unroll=_UNROLL)
            def _(i):
                sl = pl.ds(i, _LANES)
                ob[sl] = plsc.load_gather(row, [ib[sl]])

        @pl.loop(0, d_per)
        def _(j):
            d = t * d_per + j
            pltpu.make_async_copy(et_hbm.at[d], row, sem_row).start()
            pltpu.make_async_copy(it_hbm.at[0], ib0, sem_i0).start()
            pltpu.make_async_copy(it_hbm.at[1], ib1, sem_i1).start()
            pltpu.make_async_copy(et_hbm.at[d], row, sem_row).wait()

            @pl.loop(0, H // 2)
            def _(hh):
                h0 = 2 * hh
                h1 = h0 + 1

                pltpu.make_async_copy(it_hbm.at[h0], ib0, sem_i0).wait()

                @pl.when(hh > 0)
                def _():
                    pltpu.make_async_copy(ob0, o_hbm.at[h0 - 2, d],
                                          sem_o0).wait()

                gather_into(ob0, ib0)
                pltpu.make_async_copy(ob0, o_hbm.at[h0, d], sem_o0).start()

                @pl.when(h0 + 2 < H)
                def _():
                    pltpu.make_async_copy(it_hbm.at[h0 + 2], ib0,
                                          sem_i0).start()

                pltpu.make_async_copy(it_hbm.at[h1], ib1, sem_i1).wait()

                @pl.when(hh > 0)
                def _():
                    pltpu.make_async_copy(ob1, o_hbm.at[h1 - 2, d],
                                          sem_o1).wait()

                gather_into(ob1, ib1)
                pltpu.make_async_copy(ob1, o_hbm.at[h1, d], sem_o1).start()

                @pl.when(h1 + 2 < H)
                def _():
                    pltpu.make_async_copy(it_hbm.at[h1 + 2], ib1,
                                          sem_i1).start()

            pltpu.make_async_copy(ob0, o_hbm.at[H - 2, d], sem_o0).wait()
            pltpu.make_async_copy(ob1, o_hbm.at[H - 1, d], sem_o1).wait()

    out = gather_kernel(E_T, idx_T)
    return jnp.transpose(out, (2, 0, 1))
